# Initial kernel scaffold; baseline (speedup 1.0000x reference)
import jax
import jax.numpy as jnp
from jax.experimental import pallas as pl


def kernel(x, edge_index, Wl1, bl1, Wr1, Wl2, bl2, Wr2, Wlin, blin):
    raise NotImplementedError("write your pallas kernel here")


# speedup vs baseline: 4.7928x; 4.7928x over previous
import jax
import jax.numpy as jnp
from jax import lax
from jax.experimental import pallas as pl
from jax.experimental.pallas import tpu as pltpu
from jax.experimental.pallas import tpu_sc as plsc

_NC = 2
_NS = 16
_NW = _NC * _NS


def _pick_chunk(e_per_tile, cap=80):
  c = 8
  for cand in range(8, cap + 1, 8):
    if e_per_tile % cand == 0:
      c = cand
  return c


def _make_agg(N, E, W):
  e_per_tile = E // _NW
  assert e_per_tile * _NW == E
  C = _pick_chunk(e_per_tile)
  n_chunks = e_per_tile // C
  RS = 8 * ((N + 8 * _NS - 1) // (8 * _NS))
  RS_LAST = N - RS * (_NS - 1)
  assert 0 < RS_LAST <= RS

  mesh = plsc.VectorSubcoreMesh(core_axis_name="c", subcore_axis_name="s")

  def body(x_hbm, src_hbm, dst_hbm, z_hbm, out_hbm,
           acc, src_v, dst_v, rows, sem):
    cid = lax.axis_index("c")
    sid = lax.axis_index("s")
    wid = sid * _NC + cid
    base0 = wid * e_per_tile
    r0 = sid * RS

    def stripe_chunks(L, fn):
      off = 0
      while off < L:
        ln = min(C, L - off)
        fn(off, ln)
        off += ln

    def per_stripe(fn):
      @pl.when(sid != _NS - 1)
      def _():
        fn(RS)

      @pl.when(sid == _NS - 1)
      def _():
        fn(RS_LAST)

    pltpu.sync_copy(z_hbm, rows)
    per_stripe(lambda L: stripe_chunks(L, lambda off, ln: pltpu.sync_copy(
        rows.at[pl.ds(0, ln)], acc.at[pl.ds(r0 + off, ln)])))
    plsc.subcore_barrier()

    def chunk(i, carry):
      off = base0 + i * C
      pltpu.sync_copy(src_hbm.at[pl.ds(off, C)], src_v)
      pltpu.sync_copy(dst_hbm.at[pl.ds(off, C)], dst_v)
      pltpu.async_copy(x_hbm.at[src_v], rows, sem).wait()
      pltpu.sync_copy(rows, acc.at[dst_v], add=True)
      return carry

    lax.fori_loop(0, n_chunks, chunk, 0)
    plsc.subcore_barrier()

    h0 = cid * N + r0

    def wb(L):
      def piece(off, ln):
        pltpu.sync_copy(acc.at[pl.ds(r0 + off, ln)], rows.at[pl.ds(0, ln)])
        pltpu.sync_copy(rows.at[pl.ds(0, ln)], out_hbm.at[pl.ds(h0 + off, ln)])
      stripe_chunks(L, piece)

    per_stripe(wb)

  return pl.kernel(
      body,
      out_type=[jax.ShapeDtypeStruct((_NC * N, W), jnp.float32)],
      mesh=mesh,
      scratch_types=[
          pltpu.VMEM_SHARED((N, W), jnp.float32),
          pltpu.VMEM((C,), jnp.int32),
          pltpu.VMEM((C,), jnp.int32),
          pltpu.VMEM((C, W), jnp.float32),
          pltpu.SemaphoreType.DMA,
      ],
  )


def _make_cnt(N, E, W):
  e_per_tile = E // _NW
  C = _pick_chunk(e_per_tile)
  n_chunks = e_per_tile // C
  RS = 8 * ((N + 8 * _NS - 1) // (8 * _NS))
  RS_LAST = N - RS * (_NS - 1)

  mesh = plsc.VectorSubcoreMesh(core_axis_name="c", subcore_axis_name="s")

  def body(dst_hbm, z_hbm, ones_hbm, out_hbm, acc, dst_v, rows, ones_v):
    cid = lax.axis_index("c")
    sid = lax.axis_index("s")
    wid = sid * _NC + cid
    base0 = wid * e_per_tile
    r0 = sid * RS

    def stripe_chunks(L, fn):
      off = 0
      while off < L:
        ln = min(C, L - off)
        fn(off, ln)
        off += ln

    def per_stripe(fn):
      @pl.when(sid != _NS - 1)
      def _():
        fn(RS)

      @pl.when(sid == _NS - 1)
      def _():
        fn(RS_LAST)

    pltpu.sync_copy(z_hbm, rows)
    pltpu.sync_copy(ones_hbm, ones_v)
    per_stripe(lambda L: stripe_chunks(L, lambda off, ln: pltpu.sync_copy(
        rows.at[pl.ds(0, ln)], acc.at[pl.ds(r0 + off, ln)])))
    plsc.subcore_barrier()

    def chunk(i, carry):
      pltpu.sync_copy(dst_hbm.at[pl.ds(base0 + i * C, C)], dst_v)
      pltpu.sync_copy(ones_v, acc.at[dst_v], add=True)
      return carry

    lax.fori_loop(0, n_chunks, chunk, 0)
    plsc.subcore_barrier()

    h0 = cid * N + r0

    def wb(L):
      def piece(off, ln):
        pltpu.sync_copy(acc.at[pl.ds(r0 + off, ln)], rows.at[pl.ds(0, ln)])
        pltpu.sync_copy(rows.at[pl.ds(0, ln)], out_hbm.at[pl.ds(h0 + off, ln)])
      stripe_chunks(L, piece)

    per_stripe(wb)

  return pl.kernel(
      body,
      out_type=[jax.ShapeDtypeStruct((_NC * N, W), jnp.float32)],
      mesh=mesh,
      scratch_types=[
          pltpu.VMEM_SHARED((N, W), jnp.float32),
          pltpu.VMEM((C,), jnp.int32),
          pltpu.VMEM((C, W), jnp.float32),
          pltpu.VMEM((C, W), jnp.float32),
      ],
  )


def _mm_t(a, w):
  return lax.dot_general(a, w, (((1,), (1,)), ((), ())),
                         preferred_element_type=jnp.float32)


def _tc_layer1(acc2, cnts, x, Wl, bl, Wr):
  N, D = x.shape
  H = Wl.shape[0]

  def body(acc_ref, cnt_ref, x_ref, wl_ref, bl_ref, wr_ref, o_ref):
    s = acc_ref[0] + acc_ref[1]
    c = cnt_ref[0, :, 0:1] + cnt_ref[1, :, 0:1]
    mean = s / jnp.maximum(c, 1.0)
    t = _mm_t(mean, wl_ref[...]) + bl_ref[...] + _mm_t(x_ref[...], wr_ref[...])
    o_ref[...] = jnp.maximum(t, 0.0)

  return pl.pallas_call(
      body,
      out_shape=jax.ShapeDtypeStruct((N, H), jnp.float32),
  )(acc2, cnts, x, Wl, bl.reshape(1, -1), Wr)


def _tc_layer2(acc2, cnts, h, Wl, bl, Wr, Wlin, blin):
  N, H = h.shape
  O = Wlin.shape[0]

  def body(acc_ref, cnt_ref, h_ref, wl_ref, bl_ref, wr_ref, wlin_ref,
           blin_ref, h2_ref, xp_ref):
    s = acc_ref[0] + acc_ref[1]
    c = cnt_ref[0, :, 0:1] + cnt_ref[1, :, 0:1]
    mean = s / jnp.maximum(c, 1.0)
    h2 = _mm_t(mean, wl_ref[...]) + bl_ref[...] + _mm_t(h_ref[...], wr_ref[...])
    h2_ref[...] = h2
    xp_ref[...] = _mm_t(jnp.maximum(h2, 0.0), wlin_ref[...]) + blin_ref[...]

  return pl.pallas_call(
      body,
      out_shape=[
          jax.ShapeDtypeStruct((N, H), jnp.float32),
          jax.ShapeDtypeStruct((N, O), jnp.float32),
      ],
  )(acc2, cnts, h, Wl, bl.reshape(1, -1), Wr, Wlin, blin.reshape(1, -1))


def kernel(x, edge_index, Wl1, bl1, Wr1, Wl2, bl2, Wr2, Wlin, blin):
  N, D = x.shape
  H = Wl1.shape[0]
  E = edge_index.shape[1]
  src = edge_index[0]
  dst = edge_index[1]
  e_per_tile = E // _NW
  C = _pick_chunk(e_per_tile)
  z_d = jnp.zeros((C, D), jnp.float32)
  z_h = jnp.zeros((C, H), jnp.float32)
  ones_d = jnp.ones((C, D), jnp.float32)

  (acc1,) = _make_agg(N, E, D)(x, src, dst, z_d)
  acc1 = acc1.reshape(_NC, N, D)
  (cnts,) = _make_cnt(N, E, D)(dst, z_d, ones_d)
  cnts = cnts.reshape(_NC, N, D)
  h = _tc_layer1(acc1, cnts, x, Wl1, bl1, Wr1)
  (acc2,) = _make_agg(N, E, H)(h, src, dst, z_h)
  acc2 = acc2.reshape(_NC, N, H)
  h2, x_post = _tc_layer2(acc2, cnts, h, Wl2, bl2, Wr2, Wlin, blin)
  return (h2, x_post)
